# Initial kernel scaffold; baseline (speedup 1.0000x reference)
import jax
import jax.numpy as jnp
from jax.experimental import pallas as pl


def kernel(node_features, relation_features, adj, w0, b0, w1, b1):
    raise NotImplementedError("write your pallas kernel here")


# speedup vs baseline: 3.6349x; 3.6349x over previous
import jax
import jax.numpy as jnp
from jax.experimental import pallas as pl
from jax.experimental.pallas import tpu as pltpu

_CD = jnp.bfloat16


def _fused_rgcn_kernel(x_ref, adj_ref, wx0_ref, relp0_ref, b0_ref,
                       wx1_ref, relp1_ref, b1_ref, out_ref):
    R = adj_ref.shape[1]
    D = b0_ref.shape[1]
    adj_c = [adj_ref[0, r].astype(_CD) for r in range(R)]

    h = x_ref[0]
    for wx_ref, relp_ref, b_ref in ((wx0_ref, relp0_ref, b0_ref),
                                    (wx1_ref, relp1_ref, b1_ref)):
        y = jnp.dot(h.astype(_CD), wx_ref[...],
                    preferred_element_type=jnp.float32)
        y = (y + relp_ref[0]).astype(_CD)
        acc = jnp.dot(adj_c[0], y[:, :D], preferred_element_type=jnp.float32)
        for r in range(1, R):
            acc += jnp.dot(adj_c[r], y[:, r * D:(r + 1) * D],
                           preferred_element_type=jnp.float32)
        h = jnp.maximum(acc + b_ref[...], 0.0)
    out_ref[0] = h


def _prep_layer(w, rel, in_dim):
    B, R, L = rel.shape
    Dout = w.shape[0]
    K = in_dim + L
    w_all = jnp.transpose(w).reshape(R, K, Dout)
    wx = jnp.transpose(w_all[:, :in_dim, :], (1, 0, 2)).reshape(in_dim, R * Dout)
    wrel = w_all[:, in_dim:, :]
    relp = jnp.einsum("brl,rld->brd", rel, wrel).reshape(B, 1, R * Dout)
    return wx.astype(_CD), relp, Dout


def kernel(node_features, relation_features, adj, w0, b0, w1, b1):
    B, N, Din = node_features.shape
    _, R, _ = relation_features.shape

    wx0, relp0, D0 = _prep_layer(w0, relation_features, Din)
    wx1, relp1, D1 = _prep_layer(w1, relation_features, D0)
    b0_2 = b0.reshape(1, D0).astype(jnp.float32)
    b1_2 = b1.reshape(1, D1).astype(jnp.float32)

    return pl.pallas_call(
        _fused_rgcn_kernel,
        out_shape=jax.ShapeDtypeStruct((B, N, D1), node_features.dtype),
        grid=(B,),
        in_specs=[
            pl.BlockSpec((1, N, Din), lambda b: (b, 0, 0)),
            pl.BlockSpec((1, R, N, N), lambda b: (b, 0, 0, 0)),
            pl.BlockSpec((Din, R * D0), lambda b: (0, 0)),
            pl.BlockSpec((1, 1, R * D0), lambda b: (b, 0, 0)),
            pl.BlockSpec((1, D0), lambda b: (0, 0)),
            pl.BlockSpec((D0, R * D1), lambda b: (0, 0)),
            pl.BlockSpec((1, 1, R * D1), lambda b: (b, 0, 0)),
            pl.BlockSpec((1, D1), lambda b: (0, 0)),
        ],
        out_specs=pl.BlockSpec((1, N, D1), lambda b: (b, 0, 0)),
        compiler_params=pltpu.CompilerParams(
            dimension_semantics=("parallel",),
            vmem_limit_bytes=int((64 << 20) * 0.75)),
    )(node_features, adj, wx0, relp0, b0_2, wx1, relp1, b1_2)
